# speedup vs baseline: 3.3069x; 3.3069x over previous
import functools

import jax
import jax.numpy as jnp
from jax import lax
from jax.experimental import pallas as pl
from jax.experimental.pallas import tpu as pltpu, tpu_sc as plsc

NC, NS = 2, 16
NW = NC * NS


def _make_gather(V, D, B):
    assert D % 16 == 0 and B % (8 * NW) == 0
    b_per_w = B // NW
    CH = 400
    assert b_per_w % CH == 0
    n_chunks = b_per_w // CH

    mesh = plsc.VectorSubcoreMesh(core_axis_name="c", subcore_axis_name="s")

    @functools.partial(
        pl.kernel,
        out_type=jax.ShapeDtypeStruct((B, D), jnp.float32),
        mesh=mesh,
        scratch_types=[
            pltpu.VMEM((b_per_w,), jnp.int32),
            pltpu.VMEM((CH, D), jnp.float32),
            pltpu.VMEM((CH, D), jnp.float32),
            pltpu.SemaphoreType.DMA,
            pltpu.SemaphoreType.DMA,
        ],
    )
    def gather(table_hbm, idx_hbm, out_hbm, idx_v, buf0, buf1, sem_g, sem_s):
        wid = lax.axis_index("s") * NC + lax.axis_index("c")
        base = wid * b_per_w
        pltpu.sync_copy(idx_hbm.at[pl.ds(base, b_per_w)], idx_v)

        bufs = (buf0, buf1)

        def start_gather(c):
            return pltpu.async_copy(
                table_hbm.at[idx_v.at[pl.ds(c * CH, CH)]], bufs[c % 2], sem_g)

        def start_scatter(c):
            return pltpu.async_copy(
                bufs[c % 2], out_hbm.at[pl.ds(base + c * CH, CH)], sem_s)

        g = start_gather(0)
        scat = None
        for c in range(n_chunks):
            g.wait()
            if c + 1 < n_chunks:
                if scat is not None:
                    scat.wait()
                g = start_gather(c + 1)
            elif scat is not None:
                scat.wait()
            scat = start_scatter(c)
        scat.wait()

    return gather


def kernel(tokens, table):
    B0, S = tokens.shape
    V, D = table.shape
    idx = tokens.reshape(B0 * S).astype(jnp.int32)
    out = _make_gather(V, D, B0 * S)(table, idx)
    return out.reshape(B0, S, D)


# speedup vs baseline: 3.3600x; 1.0161x over previous
import functools

import jax
import jax.numpy as jnp
from jax import lax
from jax.experimental import pallas as pl
from jax.experimental.pallas import tpu as pltpu, tpu_sc as plsc

NC, NS = 2, 16
NW = NC * NS


def _make_gather(V, D, B):
    assert D % 16 == 0 and B % (8 * NW) == 0
    b_per_w = B // NW
    CH = 200
    assert b_per_w % (4 * CH) == 0
    n_groups = b_per_w // (4 * CH)

    mesh = plsc.VectorSubcoreMesh(core_axis_name="c", subcore_axis_name="s")

    @functools.partial(
        pl.kernel,
        out_type=jax.ShapeDtypeStruct((B, D), jnp.float32),
        mesh=mesh,
        scratch_types=[
            pltpu.VMEM((b_per_w,), jnp.int32),
            pltpu.VMEM((CH, D), jnp.float32),
            pltpu.VMEM((CH, D), jnp.float32),
            pltpu.VMEM((CH, D), jnp.float32),
            pltpu.VMEM((CH, D), jnp.float32),
            pltpu.SemaphoreType.DMA,
            pltpu.SemaphoreType.DMA,
        ],
    )
    def gather(table_hbm, idx_hbm, out_hbm, idx_v, b0, b1, b2, b3,
               sem_g, sem_s):
        bufs = (b0, b1, b2, b3)
        wid = lax.axis_index("s") * NC + lax.axis_index("c")
        base = wid * b_per_w
        pltpu.sync_copy(idx_hbm.at[pl.ds(base, b_per_w)], idx_v)

        def fire_gather(off, buf):
            pltpu.async_copy(table_hbm.at[idx_v.at[pl.ds(off, CH)]], buf,
                             sem_g)

        def wait_gather(buf):
            pltpu.make_async_copy(table_hbm.at[pl.ds(0, CH)], buf,
                                  sem_g).wait()

        def fire_scatter(off, buf):
            pltpu.async_copy(buf, out_hbm.at[pl.ds(base + off, CH)], sem_s)

        def wait_scatter(buf):
            pltpu.make_async_copy(buf, out_hbm.at[pl.ds(base, CH)],
                                  sem_s).wait()

        def step(c_off, b, do_wait_scatter, fire_next):
            wait_gather(bufs[b])
            fire_scatter(c_off, bufs[b])
            if fire_next:
                if do_wait_scatter:
                    wait_scatter(bufs[(b + 2) % 4])
                fire_gather(c_off + 2 * CH, bufs[(b + 2) % 4])

        fire_gather(0, bufs[0])
        fire_gather(CH, bufs[1])

        for b in range(4):
            step(b * CH, b, b >= 2, True)

        @pl.loop(1, n_groups - 1)
        def _(g):
            goff = pl.multiple_of(g * (4 * CH), 8)
            for b in range(4):
                step(goff + b * CH, b, True, True)

        last = (n_groups - 1) * 4 * CH
        for b in range(4):
            step(last + b * CH, b, True, b < 2)

        for b in range(4):
            wait_scatter(bufs[b])

    return gather


def kernel(tokens, table):
    B0, S = tokens.shape
    V, D = table.shape
    idx = tokens.reshape(B0 * S).astype(jnp.int32)
    out = _make_gather(V, D, B0 * S)(table, idx)
    return out.reshape(B0, S, D)


# speedup vs baseline: 5.9151x; 1.7604x over previous
import functools

import jax
import jax.numpy as jnp
from jax import lax
from jax.experimental import pallas as pl
from jax.experimental.pallas import tpu as pltpu, tpu_sc as plsc

NC, NS = 2, 16
NW = NC * NS
Q = 8


def _make_gather(V, D, B0, S):
    assert B0 % NW == 0 and D % 16 == 0
    seq_per_w = B0 // NW
    assert seq_per_w % (2 * Q) == 0
    n_chunks = seq_per_w // Q

    mesh = plsc.VectorSubcoreMesh(core_axis_name="c", subcore_axis_name="s")

    @functools.partial(
        pl.kernel,
        out_type=jax.ShapeDtypeStruct((B0, S, D), jnp.float32),
        mesh=mesh,
        scratch_types=[
            pltpu.VMEM((seq_per_w, S), jnp.int32),
            pltpu.VMEM((Q, S, D), jnp.float32),
            pltpu.VMEM((Q, S, D), jnp.float32),
            pltpu.SemaphoreType.DMA,
            pltpu.SemaphoreType.DMA,
        ],
    )
    def gather(table_hbm, tok_hbm, out_hbm, idx_v, buf0, buf1,
               sem_g, sem_s):
        bufs = (buf0, buf1)
        wid = lax.axis_index("s") * NC + lax.axis_index("c")
        s_base = wid * seq_per_w
        pltpu.sync_copy(tok_hbm.at[pl.ds(s_base, seq_per_w)], idx_v)

        def fire_gathers(c, buf):
            for q in range(Q):
                pltpu.async_copy(table_hbm.at[idx_v.at[c * Q + q]],
                                 buf.at[q], sem_g)

        def wait_gathers(buf):
            for q in range(Q):
                pltpu.make_async_copy(table_hbm.at[idx_v.at[0]],
                                      buf.at[q], sem_g).wait()

        def fire_scatter(c, buf):
            pltpu.async_copy(buf, out_hbm.at[pl.ds(s_base + c * Q, Q)],
                             sem_s)

        def wait_scatter(buf):
            pltpu.make_async_copy(buf, out_hbm.at[pl.ds(s_base, Q)],
                                  sem_s).wait()

        def do_step(c, b, nxt_b, first, last):
            wait_gathers(bufs[b])
            if not last:
                if not first:
                    wait_scatter(bufs[nxt_b])
                fire_gathers(c + 1, bufs[nxt_b])
            fire_scatter(c, bufs[b])

        fire_gathers(0, bufs[0])

        do_step(0, 0, 1, True, False)

        @pl.loop(0, (n_chunks - 2) // 2)
        def _(g):
            c1 = 2 * g + 1
            do_step(c1, 1, 0, False, False)
            do_step(c1 + 1, 0, 1, False, False)

        do_step(n_chunks - 1, 1, 0, False, True)

        wait_scatter(bufs[0])
        wait_scatter(bufs[1])

    return gather


def kernel(tokens, table):
    B0, S = tokens.shape
    V, D = table.shape
    out = _make_gather(V, D, B0, S)(table, tokens.astype(jnp.int32))
    return out


# speedup vs baseline: 8.6198x; 1.4572x over previous
import functools

import jax
import jax.numpy as jnp
from jax import lax
from jax.experimental import pallas as pl
from jax.experimental.pallas import tpu as pltpu, tpu_sc as plsc

NC, NS = 2, 16
NW = NC * NS


def _make_gather(V, D, B0, S):
    assert B0 % NW == 0 and D % 16 == 0 and S % 2 == 0
    n_per_w = B0 // NW

    mesh = plsc.VectorSubcoreMesh(core_axis_name="c", subcore_axis_name="s")

    @functools.partial(
        pl.kernel,
        out_type=jax.ShapeDtypeStruct((S, B0, D), jnp.float32),
        mesh=mesh,
        scratch_types=[
            pltpu.VMEM((S, n_per_w), jnp.int32),
            pltpu.VMEM((n_per_w, D), jnp.float32),
            pltpu.VMEM((n_per_w, D), jnp.float32),
            pltpu.SemaphoreType.DMA,
            pltpu.SemaphoreType.DMA,
        ],
    )
    def gather(table_hbm, tok_hbm, out_hbm, idx_v, buf0, buf1,
               sem_g, sem_s):
        bufs = (buf0, buf1)
        wid = lax.axis_index("s") * NC + lax.axis_index("c")
        col = wid * n_per_w
        pltpu.sync_copy(tok_hbm.at[:, pl.ds(col, n_per_w)], idx_v)

        def fire_gather(s, buf):
            pltpu.async_copy(table_hbm.at[idx_v.at[s]], buf, sem_g)

        def wait_gather(buf):
            pltpu.make_async_copy(table_hbm.at[idx_v.at[0]], buf,
                                  sem_g).wait()

        def fire_scatter(s, buf):
            pltpu.async_copy(buf, out_hbm.at[s, pl.ds(col, n_per_w)],
                             sem_s)

        def wait_scatter(buf):
            pltpu.make_async_copy(buf, out_hbm.at[0, pl.ds(col, n_per_w)],
                                  sem_s).wait()

        def step(s, b, first, last):
            wait_gather(bufs[b])
            if not last:
                if not first:
                    wait_scatter(bufs[1 - b])
                fire_gather(s + 1, bufs[1 - b])
            fire_scatter(s, bufs[b])

        fire_gather(0, bufs[0])
        step(0, 0, True, False)

        @pl.loop(0, (S - 2) // 2)
        def _(g):
            s1 = 2 * g + 1
            step(s1, 1, False, False)
            step(s1 + 1, 0, False, False)

        step(S - 1, 1, False, True)

        wait_scatter(bufs[0])
        wait_scatter(bufs[1])

    return gather


def kernel(tokens, table):
    B0, S = tokens.shape
    V, D = table.shape
    tok_t = jnp.transpose(tokens.astype(jnp.int32))
    out_t = _make_gather(V, D, B0, S)(table, tok_t)
    return jnp.transpose(out_t, (1, 0, 2))


# speedup vs baseline: 10.8247x; 1.2558x over previous
import functools

import jax
import jax.numpy as jnp
from jax import lax
from jax.experimental import pallas as pl
from jax.experimental.pallas import tpu as pltpu, tpu_sc as plsc

NC, NS = 2, 16
NW = NC * NS


def _make_gather(V, D, B0, S):
    assert B0 % NW == 0 and D % 16 == 0 and S % 2 == 0
    n_per_w = B0 // NW

    mesh = plsc.VectorSubcoreMesh(core_axis_name="c", subcore_axis_name="s")

    @functools.partial(
        pl.kernel,
        out_type=jax.ShapeDtypeStruct((S, B0, D), jnp.float32),
        mesh=mesh,
        scratch_types=[
            pltpu.VMEM((S, n_per_w), jnp.int32),
            pltpu.VMEM((n_per_w, D), jnp.float32),
            pltpu.VMEM((n_per_w, D), jnp.float32),
            pltpu.VMEM((n_per_w, D), jnp.float32),
            pltpu.VMEM((n_per_w, D), jnp.float32),
            pltpu.SemaphoreType.DMA,
            pltpu.SemaphoreType.DMA,
        ],
    )
    def gather(table_hbm, tok_hbm, out_hbm, idx_v, buf0, buf1, buf2, buf3,
               sem_g, sem_s):
        bufs = (buf0, buf1, buf2, buf3)
        wid = lax.axis_index("s") * NC + lax.axis_index("c")
        col = wid * n_per_w
        pltpu.sync_copy(tok_hbm.at[:, pl.ds(col, n_per_w)], idx_v)

        def fire_gather(s, buf):
            pltpu.async_copy(table_hbm.at[idx_v.at[s]], buf, sem_g)

        def wait_gather(buf):
            pltpu.make_async_copy(table_hbm.at[idx_v.at[0]], buf,
                                  sem_g).wait()

        def fire_scatter(s, buf):
            pltpu.async_copy(buf, out_hbm.at[s, pl.ds(col, n_per_w)],
                             sem_s)

        def wait_scatter(buf):
            pltpu.make_async_copy(buf, out_hbm.at[0, pl.ds(col, n_per_w)],
                                  sem_s).wait()

        K = 3

        def step(s, b, wait_s, fire_n):
            wait_gather(bufs[b])
            fire_scatter(s, bufs[b])
            if fire_n:
                if wait_s:
                    wait_scatter(bufs[(b + K) % 4])
                fire_gather(s + K, bufs[(b + K) % 4])

        for j in range(K):
            fire_gather(j, bufs[j])

        for s in range(4):
            step(s, s % 4, s >= 1, True)

        @pl.loop(1, (S - 4) // 4)
        def _(g):
            s0 = 4 * g
            for b in range(4):
                step(s0 + b, b, True, True)

        for s in range(S - 6, S):
            step(s, s % 4, s + K < S, s + K < S)

        for b in range(4):
            wait_scatter(bufs[b])

    return gather


def kernel(tokens, table):
    B0, S = tokens.shape
    V, D = table.shape
    tok_t = jnp.transpose(tokens.astype(jnp.int32))
    out_t = _make_gather(V, D, B0, S)(table, tok_t)
    return jnp.transpose(out_t, (1, 0, 2))
